# baseline (speedup 1.0000x reference)
import functools

import jax
import jax.numpy as jnp
from jax import lax
from jax.experimental import pallas as pl
from jax.experimental.pallas import tpu as pltpu
from jax.experimental.pallas import tpu_sc as plsc

E = 8
K = 2
H = 1024
I = 2048
T = 2048

TB = 256
NB = T // TB
RT = 256
JS = 2
R = 4416


def _route_body(gate_ref, x_ref, wts_ref, p0_ref, p1_ref, cnt_ref,
                idx_s, wts_s, cnt_s, run_s, start_s):
    j = pl.program_id(0)
    i = pl.program_id(1)
    ei = lax.broadcasted_iota(jnp.int32, (TB, E), 1)

    @pl.when(j == 0)
    def _phase0():
        x = x_ref[...]
        g = gate_ref[...]
        logits = lax.dot_general(
            x, g, (((1,), (1,)), ((), ())),
            preferred_element_type=jnp.float32,
        )
        m1 = jnp.max(logits, axis=1, keepdims=True)
        a1 = jnp.min(jnp.where(logits == m1, ei, E), axis=1, keepdims=True)
        masked = jnp.where(ei == a1, -jnp.inf, logits)
        m2 = jnp.max(masked, axis=1, keepdims=True)
        a2 = jnp.min(jnp.where(masked == m2, ei, E), axis=1, keepdims=True)
        t = jnp.exp(m2 - m1)
        w0 = 1.0 / (1.0 + t)
        w1v = t / (1.0 + t)
        idx = jnp.where(ei == 0, a1, jnp.where(ei == 1, a2, 0))
        wts = jnp.where(ei == 0, w0, jnp.where(ei == 1, w1v, 0.0))
        wts_ref[...] = wts
        idx_s[pl.ds(i, 1)] = idx[None]
        wts_s[pl.ds(i, 1)] = wts[None]
        oh = ((ei == a1) | (ei == a2)).astype(jnp.float32)
        cnt = jnp.sum(oh, axis=0, keepdims=True)
        cnt_s[pl.ds(i, 1)] = cnt[None]
        cnt_ref[...] = cnt.astype(jnp.int32)[None]
        p0_ref[...] = jnp.zeros((TB, 1), jnp.int32)
        p1_ref[...] = jnp.zeros((TB, 1), jnp.int32)

    @pl.when(j == 1)
    def _phase1():
        @pl.when(i == 0)
        def _init():
            tot = jnp.sum(cnt_s[...], axis=0)
            padded = jnp.floor((tot + 7.0) / 8.0) * 8.0
            eu = lax.broadcasted_iota(jnp.int32, (E, E), 0)
            ev = lax.broadcasted_iota(jnp.int32, (E, E), 1)
            m = (eu < ev).astype(jnp.float32)
            start_s[...] = lax.dot_general(
                padded, m, (((1,), (0,)), ((), ())),
                preferred_element_type=jnp.float32)
            run_s[...] = jnp.zeros((1, E), jnp.float32)

        idx = idx_s[pl.ds(i, 1)][0]
        a1 = idx[:, 0:1]
        a2 = idx[:, 1:2]
        oh0 = (ei == a1).astype(jnp.float32)
        oh1 = (ei == a2).astype(jnp.float32)
        ri = lax.broadcasted_iota(jnp.int32, (TB, TB), 0)
        ci = lax.broadcasted_iota(jnp.int32, (TB, TB), 1)
        tri = (ri >= ci).astype(jnp.bfloat16)
        inc0 = jnp.dot(tri, oh0.astype(jnp.bfloat16),
                       preferred_element_type=jnp.float32)
        inc1 = jnp.dot(tri, oh1.astype(jnp.bfloat16),
                       preferred_element_type=jnp.float32)
        excl = (inc0 - oh0) + (inc1 - oh1)
        base = start_s[...] + run_s[...] + excl
        p0 = jnp.sum(base * oh0, axis=1, keepdims=True)
        p1 = jnp.sum(base * oh1, axis=1, keepdims=True)
        p0_ref[...] = p0.astype(jnp.int32)
        p1_ref[...] = p1.astype(jnp.int32)
        wts_ref[...] = wts_s[pl.ds(i, 1)][0]
        cnt_ref[...] = cnt_s[pl.ds(i, 1)].astype(jnp.int32)
        run_s[...] = run_s[...] + cnt_s[pl.ds(i, 1)][0]


def _route(x, gate_w):
    return pl.pallas_call(
        _route_body,
        grid=(2, NB),
        in_specs=[
            pl.BlockSpec((E, H), lambda j, i: (0, 0)),
            pl.BlockSpec((TB, H), lambda j, i: (i * (1 - j), 0)),
        ],
        out_specs=[
            pl.BlockSpec((TB, E), lambda j, i: (i, 0)),
            pl.BlockSpec((TB, 1), lambda j, i: (i, 0)),
            pl.BlockSpec((TB, 1), lambda j, i: (i, 0)),
            pl.BlockSpec((1, 1, E), lambda j, i: (i, 0, 0)),
        ],
        out_shape=[
            jax.ShapeDtypeStruct((T, E), jnp.float32),
            jax.ShapeDtypeStruct((T, 1), jnp.int32),
            jax.ShapeDtypeStruct((T, 1), jnp.int32),
            jax.ShapeDtypeStruct((NB, 1, E), jnp.int32),
        ],
        scratch_shapes=[
            pltpu.VMEM((NB, TB, E), jnp.int32),
            pltpu.VMEM((NB, TB, E), jnp.float32),
            pltpu.VMEM((NB, 1, E), jnp.float32),
            pltpu.VMEM((1, E), jnp.float32),
            pltpu.VMEM((1, E), jnp.float32),
        ],
    )(gate_w, x)


SC_NC = 2
SC_NS = 16
SC_NW = SC_NC * SC_NS
TPW = T // SC_NW
CH = 32

def _sc_mesh():
    return plsc.VectorSubcoreMesh(
        core_axis_name="c", subcore_axis_name="s",
        num_cores=SC_NC, num_subcores=SC_NS)


def _dispatch(x, pos0, pos1):
    W = x.shape[1]

    @functools.partial(
        pl.kernel,
        out_type=jax.ShapeDtypeStruct((R, W), x.dtype),
        mesh=_sc_mesh(),
        scratch_types=[
            pltpu.VMEM((CH, W), x.dtype),
            pltpu.VMEM((CH,), jnp.int32),
            pltpu.VMEM((CH,), jnp.int32),
            pltpu.SemaphoreType.DMA,
        ],
    )
    def disp(x_hbm, p0_hbm, p1_hbm, xs_hbm, rows_v, i0_v, i1_v, sem):
        wid = lax.axis_index("s") * SC_NC + lax.axis_index("c")
        base = wid * TPW
        for c in range(TPW // CH):
            b = base + c * CH
            pltpu.sync_copy(x_hbm.at[pl.ds(b, CH)], rows_v)
            pltpu.sync_copy(p0_hbm.at[pl.ds(b, CH)], i0_v)
            pltpu.sync_copy(p1_hbm.at[pl.ds(b, CH)], i1_v)
            pltpu.async_copy(rows_v, xs_hbm.at[i0_v], sem).wait()
            pltpu.async_copy(rows_v, xs_hbm.at[i1_v], sem).wait()

    return disp(x, pos0, pos1)


def _combine_gather(y_sorted, pos0, pos1):
    W = y_sorted.shape[1]

    @functools.partial(
        pl.kernel,
        out_type=[
            jax.ShapeDtypeStruct((T, W), y_sorted.dtype),
            jax.ShapeDtypeStruct((T, W), y_sorted.dtype),
        ],
        mesh=_sc_mesh(),
        scratch_types=[
            pltpu.VMEM((CH, W), y_sorted.dtype),
            pltpu.VMEM((CH, W), y_sorted.dtype),
            pltpu.VMEM((CH,), jnp.int32),
            pltpu.VMEM((CH,), jnp.int32),
            pltpu.SemaphoreType.DMA,
        ],
    )
    def comb(y_hbm, p0_hbm, p1_hbm, g0_hbm, g1_hbm, r0_v, r1_v, i0_v, i1_v, sem):
        wid = lax.axis_index("s") * SC_NC + lax.axis_index("c")
        base = wid * TPW
        for c in range(TPW // CH):
            b = base + c * CH
            pltpu.sync_copy(p0_hbm.at[pl.ds(b, CH)], i0_v)
            pltpu.sync_copy(p1_hbm.at[pl.ds(b, CH)], i1_v)
            pltpu.async_copy(y_hbm.at[i0_v], r0_v, sem).wait()
            pltpu.async_copy(y_hbm.at[i1_v], r1_v, sem).wait()
            pltpu.sync_copy(r0_v, g0_hbm.at[pl.ds(b, CH)])
            pltpu.sync_copy(r1_v, g1_hbm.at[pl.ds(b, CH)])

    return comb(y_sorted, pos0, pos1)


def _gmm_body(starts_ref, counts_ref, x_ref, w1_ref, w3_ref, w2_ref, y_ref):
    j = pl.program_id(1)
    start = starts_ref[pl.program_id(0)]
    n = counts_ref[pl.program_id(0)]
    ntiles = (n + (RT - 1)) // RT
    w1b = w1_ref[0].astype(jnp.bfloat16)
    w3b = w3_ref[0].astype(jnp.bfloat16)
    w2b = w2_ref[0].astype(jnp.bfloat16)

    def body(t, carry):
        r0 = pl.multiple_of(start + t * RT, 8)
        xt = x_ref[pl.ds(r0, RT), :]
        a = jnp.dot(xt, w1b, preferred_element_type=jnp.float32)
        b = jnp.dot(xt, w3b, preferred_element_type=jnp.float32)
        h = (a * jax.nn.sigmoid(a) * b).astype(jnp.bfloat16)
        yt = jnp.dot(h, w2b, preferred_element_type=jnp.float32)

        @pl.when(j == 0)
        def _():
            y_ref[pl.ds(r0, RT), :] = yt.astype(jnp.bfloat16)

        @pl.when(j != 0)
        def _():
            y_ref[pl.ds(r0, RT), :] += yt.astype(jnp.bfloat16)

        return carry

    jax.lax.fori_loop(0, ntiles, body, 0)


def _gmm(starts, counts, x_sorted, w1, w3, w2):
    return pl.pallas_call(
        _gmm_body,
        grid=(E, JS),
        in_specs=[
            pl.BlockSpec(memory_space=pltpu.SMEM),
            pl.BlockSpec(memory_space=pltpu.SMEM),
            pl.BlockSpec((R, H), lambda e, j: (0, 0)),
            pl.BlockSpec((1, H, I // JS), lambda e, j: (e, 0, j)),
            pl.BlockSpec((1, H, I // JS), lambda e, j: (e, 0, j)),
            pl.BlockSpec((1, I // JS, H), lambda e, j: (e, j, 0)),
        ],
        out_specs=pl.BlockSpec((R, H), lambda e, j: (0, 0)),
        out_shape=jax.ShapeDtypeStruct((R, H), jnp.bfloat16),
    )(starts, counts, x_sorted, w1, w3, w2)


def _combine_body(g0_ref, g1_ref, w0_ref, w1_ref, o_ref):
    o_ref[...] = (g0_ref[...].astype(jnp.float32) * w0_ref[...]
                  + g1_ref[...].astype(jnp.float32) * w1_ref[...])


def _combine(g0, g1, w0, w1):
    return pl.pallas_call(
        _combine_body,
        grid=(T // TB,),
        in_specs=[
            pl.BlockSpec((TB, H), lambda i: (i, 0)),
            pl.BlockSpec((TB, H), lambda i: (i, 0)),
            pl.BlockSpec((TB, 1), lambda i: (i, 0)),
            pl.BlockSpec((TB, 1), lambda i: (i, 0)),
        ],
        out_specs=pl.BlockSpec((TB, H), lambda i: (i, 0)),
        out_shape=jax.ShapeDtypeStruct((T, H), jnp.float32),
    )(g0, g1, w0, w1)


def kernel(hidden_states, gate_w, w1, w3, w2):
    x = hidden_states.reshape(T, H)
    wts, p0c, p1c, cnt_blk = _route(x, gate_w)
    pos0 = p0c.reshape(T)
    pos1 = p1c.reshape(T)
    counts = jnp.sum(cnt_blk[:, 0, :], axis=0)
    sizes_p = (counts + 7) & ~7
    starts = jnp.concatenate(
        [jnp.zeros((1,), jnp.int32), jnp.cumsum(sizes_p, dtype=jnp.int32)])

    x_i = lax.bitcast_convert_type(
        x.astype(jnp.bfloat16).reshape(T, H // 2, 2), jnp.int32)
    xs_i = _dispatch(x_i, pos0, pos1)
    x_sorted = lax.bitcast_convert_type(xs_i, jnp.bfloat16).reshape(R, H)

    y_sorted = _gmm(starts, counts, x_sorted, w1, w3, w2)

    y_i = lax.bitcast_convert_type(y_sorted.reshape(R, H // 2, 2), jnp.int32)
    g0_i, g1_i = _combine_gather(y_i, pos0, pos1)
    g0 = lax.bitcast_convert_type(g0_i, jnp.bfloat16).reshape(T, H)
    g1 = lax.bitcast_convert_type(g1_i, jnp.bfloat16).reshape(T, H)
    out = _combine(g0, g1, wts[:, 0:1], wts[:, 1:2])
    return out.reshape(hidden_states.shape)


# speedup vs baseline: 2.5391x; 2.5391x over previous
import functools

import jax
import jax.numpy as jnp
from jax import lax
from jax.experimental import pallas as pl
from jax.experimental.pallas import tpu as pltpu
from jax.experimental.pallas import tpu_sc as plsc

E = 8
K = 2
H = 1024
I = 2048
T = 2048

TB = 256
NB = T // TB
RT = 256
JS = 4
R = 4416


def _route_body(gate_ref, x_ref, wts_ref, p0_ref, p1_ref, cnt_ref,
                idx_s, wts_s, cnt_s, run_s, start_s):
    j = pl.program_id(0)
    i = pl.program_id(1)
    ei = lax.broadcasted_iota(jnp.int32, (TB, E), 1)

    @pl.when(j == 0)
    def _phase0():
        x = x_ref[...]
        g = gate_ref[...]
        logits = lax.dot_general(
            x, g, (((1,), (1,)), ((), ())),
            preferred_element_type=jnp.float32,
        )
        m1 = jnp.max(logits, axis=1, keepdims=True)
        a1 = jnp.min(jnp.where(logits == m1, ei, E), axis=1, keepdims=True)
        masked = jnp.where(ei == a1, -jnp.inf, logits)
        m2 = jnp.max(masked, axis=1, keepdims=True)
        a2 = jnp.min(jnp.where(masked == m2, ei, E), axis=1, keepdims=True)
        t = jnp.exp(m2 - m1)
        w0 = 1.0 / (1.0 + t)
        w1v = t / (1.0 + t)
        idx = jnp.where(ei == 0, a1, jnp.where(ei == 1, a2, 0))
        wts = jnp.where(ei == 0, w0, jnp.where(ei == 1, w1v, 0.0))
        wts_ref[...] = wts
        idx_s[pl.ds(i, 1)] = idx[None]
        wts_s[pl.ds(i, 1)] = wts[None]
        oh = ((ei == a1) | (ei == a2)).astype(jnp.float32)
        cnt = jnp.sum(oh, axis=0, keepdims=True)
        cnt_s[pl.ds(i, 1)] = cnt[None]
        cnt_ref[...] = cnt.astype(jnp.int32)[None]
        p0_ref[...] = jnp.zeros((TB, 1), jnp.int32)
        p1_ref[...] = jnp.zeros((TB, 1), jnp.int32)

    @pl.when(j == 1)
    def _phase1():
        @pl.when(i == 0)
        def _init():
            tot = jnp.sum(cnt_s[...], axis=0)
            padded = jnp.floor((tot + 7.0) / 8.0) * 8.0
            eu = lax.broadcasted_iota(jnp.int32, (E, E), 0)
            ev = lax.broadcasted_iota(jnp.int32, (E, E), 1)
            m = (eu < ev).astype(jnp.float32)
            start_s[...] = lax.dot_general(
                padded, m, (((1,), (0,)), ((), ())),
                preferred_element_type=jnp.float32)
            run_s[...] = jnp.zeros((1, E), jnp.float32)

        idx = idx_s[pl.ds(i, 1)][0]
        a1 = idx[:, 0:1]
        a2 = idx[:, 1:2]
        oh0 = (ei == a1).astype(jnp.float32)
        oh1 = (ei == a2).astype(jnp.float32)
        ri = lax.broadcasted_iota(jnp.int32, (TB, TB), 0)
        ci = lax.broadcasted_iota(jnp.int32, (TB, TB), 1)
        tri = (ri >= ci).astype(jnp.bfloat16)
        inc0 = jnp.dot(tri, oh0.astype(jnp.bfloat16),
                       preferred_element_type=jnp.float32)
        inc1 = jnp.dot(tri, oh1.astype(jnp.bfloat16),
                       preferred_element_type=jnp.float32)
        excl = (inc0 - oh0) + (inc1 - oh1)
        base = start_s[...] + run_s[...] + excl
        p0 = jnp.sum(base * oh0, axis=1, keepdims=True)
        p1 = jnp.sum(base * oh1, axis=1, keepdims=True)
        p0_ref[...] = p0.astype(jnp.int32)
        p1_ref[...] = p1.astype(jnp.int32)
        wts_ref[...] = wts_s[pl.ds(i, 1)][0]
        cnt_ref[...] = cnt_s[pl.ds(i, 1)].astype(jnp.int32)
        run_s[...] = run_s[...] + cnt_s[pl.ds(i, 1)][0]


def _route(x, gate_w):
    return pl.pallas_call(
        _route_body,
        grid=(2, NB),
        in_specs=[
            pl.BlockSpec((E, H), lambda j, i: (0, 0)),
            pl.BlockSpec((TB, H), lambda j, i: (i * (1 - j), 0)),
        ],
        out_specs=[
            pl.BlockSpec((TB, E), lambda j, i: (i, 0)),
            pl.BlockSpec((TB, 1), lambda j, i: (i, 0)),
            pl.BlockSpec((TB, 1), lambda j, i: (i, 0)),
            pl.BlockSpec((1, 1, E), lambda j, i: (i, 0, 0)),
        ],
        out_shape=[
            jax.ShapeDtypeStruct((T, E), jnp.float32),
            jax.ShapeDtypeStruct((T, 1), jnp.int32),
            jax.ShapeDtypeStruct((T, 1), jnp.int32),
            jax.ShapeDtypeStruct((NB, 1, E), jnp.int32),
        ],
        scratch_shapes=[
            pltpu.VMEM((NB, TB, E), jnp.int32),
            pltpu.VMEM((NB, TB, E), jnp.float32),
            pltpu.VMEM((NB, 1, E), jnp.float32),
            pltpu.VMEM((1, E), jnp.float32),
            pltpu.VMEM((1, E), jnp.float32),
        ],
    )(gate_w, x)


SC_NC = 2
SC_NS = 16
SC_NW = SC_NC * SC_NS
TPW = T // SC_NW
CH = 32

def _sc_mesh():
    return plsc.VectorSubcoreMesh(
        core_axis_name="c", subcore_axis_name="s",
        num_cores=SC_NC, num_subcores=SC_NS)


def _dispatch(x, pos0, pos1):
    W = x.shape[1]

    @functools.partial(
        pl.kernel,
        out_type=jax.ShapeDtypeStruct((R, W), x.dtype),
        mesh=_sc_mesh(),
        scratch_types=[
            pltpu.VMEM((CH, W), x.dtype),
            pltpu.VMEM((CH,), jnp.int32),
            pltpu.VMEM((CH,), jnp.int32),
            pltpu.SemaphoreType.DMA,
        ],
    )
    def disp(x_hbm, p0_hbm, p1_hbm, xs_hbm, rows_v, i0_v, i1_v, sem):
        wid = lax.axis_index("s") * SC_NC + lax.axis_index("c")
        base = wid * TPW
        for c in range(TPW // CH):
            b = base + c * CH
            pltpu.sync_copy(x_hbm.at[pl.ds(b, CH)], rows_v)
            pltpu.sync_copy(p0_hbm.at[pl.ds(b, CH)], i0_v)
            pltpu.sync_copy(p1_hbm.at[pl.ds(b, CH)], i1_v)
            pltpu.async_copy(rows_v, xs_hbm.at[i0_v], sem).wait()
            pltpu.async_copy(rows_v, xs_hbm.at[i1_v], sem).wait()

    return disp(x, pos0, pos1)


def _combine_gather(y_sorted, pos0, pos1):
    W = y_sorted.shape[1]

    @functools.partial(
        pl.kernel,
        out_type=[
            jax.ShapeDtypeStruct((T, W), y_sorted.dtype),
            jax.ShapeDtypeStruct((T, W), y_sorted.dtype),
        ],
        mesh=_sc_mesh(),
        scratch_types=[
            pltpu.VMEM((CH, W), y_sorted.dtype),
            pltpu.VMEM((CH, W), y_sorted.dtype),
            pltpu.VMEM((CH,), jnp.int32),
            pltpu.VMEM((CH,), jnp.int32),
            pltpu.SemaphoreType.DMA,
        ],
    )
    def comb(y_hbm, p0_hbm, p1_hbm, g0_hbm, g1_hbm, r0_v, r1_v, i0_v, i1_v, sem):
        wid = lax.axis_index("s") * SC_NC + lax.axis_index("c")
        base = wid * TPW
        for c in range(TPW // CH):
            b = base + c * CH
            pltpu.sync_copy(p0_hbm.at[pl.ds(b, CH)], i0_v)
            pltpu.sync_copy(p1_hbm.at[pl.ds(b, CH)], i1_v)
            pltpu.async_copy(y_hbm.at[i0_v], r0_v, sem).wait()
            pltpu.async_copy(y_hbm.at[i1_v], r1_v, sem).wait()
            pltpu.sync_copy(r0_v, g0_hbm.at[pl.ds(b, CH)])
            pltpu.sync_copy(r1_v, g1_hbm.at[pl.ds(b, CH)])

    return comb(y_sorted, pos0, pos1)


def _gmm_body(starts_ref, counts_ref, x_ref, w1_ref, w3_ref, w2_ref, y_ref):
    j = pl.program_id(1)
    start = starts_ref[pl.program_id(0)]
    n = counts_ref[pl.program_id(0)]
    ntiles = (n + (RT - 1)) // RT
    w1b = w1_ref[0].astype(jnp.bfloat16)
    w3b = w3_ref[0].astype(jnp.bfloat16)
    w2b = w2_ref[0].astype(jnp.bfloat16)

    def body(t, carry):
        r0 = pl.multiple_of(start + t * RT, 8)
        xt = x_ref[pl.ds(r0, RT), :].astype(jnp.bfloat16)
        a = jnp.dot(xt, w1b, preferred_element_type=jnp.float32)
        b = jnp.dot(xt, w3b, preferred_element_type=jnp.float32)
        h = (a * jax.nn.sigmoid(a) * b).astype(jnp.bfloat16)
        yt = jnp.dot(h, w2b, preferred_element_type=jnp.float32)

        @pl.when(j == 0)
        def _():
            y_ref[pl.ds(r0, RT), :] = yt

        @pl.when(j != 0)
        def _():
            y_ref[pl.ds(r0, RT), :] += yt

        return carry

    jax.lax.fori_loop(0, ntiles, body, 0)


def _gmm(starts, counts, x_sorted, w1, w3, w2):
    return pl.pallas_call(
        _gmm_body,
        grid=(E, JS),
        in_specs=[
            pl.BlockSpec(memory_space=pltpu.SMEM),
            pl.BlockSpec(memory_space=pltpu.SMEM),
            pl.BlockSpec((R, H), lambda e, j: (0, 0)),
            pl.BlockSpec((1, H, I // JS), lambda e, j: (e, 0, j)),
            pl.BlockSpec((1, H, I // JS), lambda e, j: (e, 0, j)),
            pl.BlockSpec((1, I // JS, H), lambda e, j: (e, j, 0)),
        ],
        out_specs=pl.BlockSpec((R, H), lambda e, j: (0, 0)),
        out_shape=jax.ShapeDtypeStruct((R, H), jnp.float32),
    )(starts, counts, x_sorted, w1, w3, w2)


def _combine_body(g0_ref, g1_ref, w0_ref, w1_ref, o_ref):
    o_ref[...] = g0_ref[...] * w0_ref[...] + g1_ref[...] * w1_ref[...]


def _combine(g0, g1, w0, w1):
    return pl.pallas_call(
        _combine_body,
        grid=(T // TB,),
        in_specs=[
            pl.BlockSpec((TB, H), lambda i: (i, 0)),
            pl.BlockSpec((TB, H), lambda i: (i, 0)),
            pl.BlockSpec((TB, 1), lambda i: (i, 0)),
            pl.BlockSpec((TB, 1), lambda i: (i, 0)),
        ],
        out_specs=pl.BlockSpec((TB, H), lambda i: (i, 0)),
        out_shape=jax.ShapeDtypeStruct((T, H), jnp.float32),
    )(g0, g1, w0, w1)


def kernel(hidden_states, gate_w, w1, w3, w2):
    x = hidden_states.reshape(T, H)
    wts, p0c, p1c, cnt_blk = _route(x, gate_w)
    pos0 = p0c.reshape(T)
    pos1 = p1c.reshape(T)
    counts = jnp.sum(cnt_blk[:, 0, :], axis=0)
    sizes_p = (counts + 7) & ~7
    starts = jnp.concatenate(
        [jnp.zeros((1,), jnp.int32), jnp.cumsum(sizes_p, dtype=jnp.int32)])

    x_sorted = _dispatch(x, pos0, pos1)
    y_sorted = _gmm(starts, counts, x_sorted, w1, w3, w2)
    g0, g1 = _combine_gather(y_sorted, pos0, pos1)
    out = _combine(g0, g1, wts[:, 0:1], wts[:, 1:2])
    return out.reshape(hidden_states.shape)
